# baseline (speedup 1.0000x reference)
import functools

import jax
import jax.numpy as jnp
from jax import lax
from jax.experimental import pallas as pl
from jax.experimental.pallas import tpu as pltpu
from jax.experimental.pallas import tpu_sc as plsc

SEQ = 200
DIM = 32
BATCH = 1024
VOCAB = 1000000
LANES = 16
NW = 32
OPAD = 1025
TB = 800
TPAD = 33
NBLK = VOCAB // TB
BLK_PER_W = NBLK // NW
_sc_params = pltpu.CompilerParams(
    use_tc_tiling_on_sc=False, needs_layout_passes=False)


def _tr_body(tok_hbm, out_hbm, stage_v, obuf_v, sem):
    cid = lax.axis_index("c")
    sid = lax.axis_index("s")
    w = sid * 2 + cid
    dlanes = lax.iota(jnp.int32, LANES)

    n_blk = jnp.where(w < NBLK - BLK_PER_W * NW, BLK_PER_W + 1, BLK_PER_W)

    def stage(i, blk):
        return pltpu.async_copy(
            tok_hbm.at[:, pl.ds(blk * TB, TB)], stage_v.at[i], sem)

    def do_blk(j, _):
        blk = w + NW * j
        i = lax.rem(j, 2)

        @pl.when(j + 1 < n_blk)
        def _():
            stage(1 - i, blk + NW)

        def tr_d(d, _):
            def tr_g(g, _):
                v = stage_v[i, d, pl.ds(g * LANES, LANES)]
                plsc.store_scatter(
                    obuf_v, [g * LANES + dlanes, jnp.full((LANES,), d, jnp.int32)], v)
                return 0

            lax.fori_loop(0, TB // LANES, tr_g, 0)
            return 0

        lax.fori_loop(0, DIM, tr_d, 0)
        pltpu.sync_copy(obuf_v.at[:, pl.ds(0, DIM)],
                        out_hbm.at[pl.ds(blk * TB, TB)])
        return 0

    stage(0, w).wait()

    def do_all(j, _):
        @pl.when(j > 0)
        def _():
            pltpu.make_async_copy(
                tok_hbm.at[:, pl.ds(0, TB)], stage_v.at[lax.rem(j, 2)], sem
            ).wait()
        do_blk(j, 0)
        return 0

    lax.fori_loop(0, n_blk, do_all, 0)


def _gather_body(idx_hbm, tok_hbm, pos_hbm, out_hbm,
                 idx_v, pos_v, rows_v, out_v, sem):
    cid = lax.axis_index("c")
    sid = lax.axis_index("s")
    w = sid * 2 + cid

    pltpu.sync_copy(pos_hbm, pos_v)
    dlanes = lax.iota(jnp.int32, LANES)

    def do_l(l):
        pltpu.sync_copy(idx_hbm.at[l], idx_v)
        copies = [
            pltpu.async_copy(
                tok_hbm.at[idx_v.at[pl.ds(j * 128, 128)]],
                rows_v.at[pl.ds(j * 128, 128)],
                sem,
            )
            for j in range(BATCH // 128)
        ]
        for cp in copies:
            cp.wait()

        p0 = plsc.load_gather(pos_v, [dlanes, jnp.full((LANES,), l, jnp.int32)])
        p1 = plsc.load_gather(
            pos_v, [dlanes + LANES, jnp.full((LANES,), l, jnp.int32)])

        def tr_j(j, _):
            v0 = rows_v[j, pl.ds(0, LANES)] + p0
            v1 = rows_v[j, pl.ds(LANES, LANES)] + p1
            jb = jnp.full((LANES,), j, jnp.int32)
            plsc.store_scatter(out_v, [dlanes, jb], v0)
            plsc.store_scatter(out_v, [dlanes + LANES, jb], v1)
            return 0

        lax.fori_loop(0, BATCH, tr_j, 0)
        pltpu.sync_copy(out_v.at[:, pl.ds(0, BATCH)], out_hbm.at[l])

    for k in range(6):
        do_l(w + NW * k)

    @pl.when(w < SEQ - 6 * NW)
    def _():
        do_l(w + NW * 6)


@jax.jit
def _run(idx_t, tok_t, pos_t):
    mesh = plsc.VectorSubcoreMesh(core_axis_name="c", subcore_axis_name="s")
    tok_rows = pl.kernel(
        _tr_body,
        out_type=jax.ShapeDtypeStruct((VOCAB, DIM), jnp.float32),
        mesh=mesh,
        scratch_types=[
            pltpu.VMEM((2, DIM, TB), jnp.float32),
            pltpu.VMEM((TB, TPAD), jnp.float32),
            pltpu.SemaphoreType.DMA,
        ],
        compiler_params=_sc_params,
    )(tok_t)
    return pl.kernel(
        _gather_body,
        out_type=jax.ShapeDtypeStruct((SEQ, DIM, BATCH), jnp.float32),
        mesh=mesh,
        scratch_types=[
            pltpu.VMEM((BATCH,), jnp.int32),
            pltpu.VMEM((DIM, SEQ), jnp.float32),
            pltpu.VMEM((BATCH, DIM), jnp.float32),
            pltpu.VMEM((DIM, OPAD), jnp.float32),
            pltpu.SemaphoreType.DMA,
        ],
        compiler_params=_sc_params,
    )(idx_t, tok_rows, pos_t)


def kernel(inputs, token_table, pos_table):
    idx_t = inputs.astype(jnp.int32).T
    out = _run(idx_t, token_table.T, pos_table.T)
    return jnp.transpose(out, (2, 0, 1))


# speedup vs baseline: 5.0500x; 5.0500x over previous
import functools

import jax
import jax.numpy as jnp
from jax import lax
from jax.experimental import pallas as pl
from jax.experimental.pallas import tpu as pltpu
from jax.experimental.pallas import tpu_sc as plsc

SEQ = 200
DIM = 32
BATCH = 1024
VOCAB = 1000000
LANES = 16
NW = 32
OPAD = 1025


def _sc_body(idx_hbm, tok_hbm, pos_hbm, out_hbm,
             idx_v, pos_v, rows_v, out_v, sem):
    cid = lax.axis_index("c")
    sid = lax.axis_index("s")
    w = sid * 2 + cid

    pltpu.sync_copy(pos_hbm, pos_v)
    dlanes = lax.iota(jnp.int32, LANES)

    def do_l(l):
        pltpu.sync_copy(idx_hbm.at[l], idx_v)
        copies = [
            pltpu.async_copy(
                tok_hbm.at[idx_v.at[pl.ds(j * 128, 128)]],
                rows_v.at[pl.ds(j * 128, 128)],
                sem,
            )
            for j in range(BATCH // 128)
        ]
        for cp in copies:
            cp.wait()

        p0 = plsc.load_gather(pos_v, [dlanes, jnp.full((LANES,), l, jnp.int32)])
        p1 = plsc.load_gather(
            pos_v, [dlanes + LANES, jnp.full((LANES,), l, jnp.int32)])

        def tr_j(j, _):
            v0 = rows_v[j, pl.ds(0, LANES)] + p0
            v1 = rows_v[j, pl.ds(LANES, LANES)] + p1
            jb = jnp.full((LANES,), j, jnp.int32)
            plsc.store_scatter(out_v, [dlanes, jb], v0)
            plsc.store_scatter(out_v, [dlanes + LANES, jb], v1)
            return 0

        lax.fori_loop(0, BATCH, tr_j, 0)
        pltpu.sync_copy(out_v.at[:, pl.ds(0, BATCH)], out_hbm.at[l])

    for k in range(6):
        do_l(w + NW * k)

    @pl.when(w < SEQ - 6 * NW)
    def _():
        do_l(w + NW * 6)


@jax.jit
def _run(idx_t, tok, pos_t):
    mesh = plsc.VectorSubcoreMesh(core_axis_name="c", subcore_axis_name="s")
    return pl.kernel(
        _sc_body,
        out_type=jax.ShapeDtypeStruct((SEQ, DIM, BATCH), jnp.float32),
        mesh=mesh,
        scratch_types=[
            pltpu.VMEM((BATCH,), jnp.int32),
            pltpu.VMEM((DIM, SEQ), jnp.float32),
            pltpu.VMEM((BATCH, DIM), jnp.float32),
            pltpu.VMEM((DIM, OPAD), jnp.float32),
            pltpu.SemaphoreType.DMA,
        ],
        compiler_params=pltpu.CompilerParams(
            use_tc_tiling_on_sc=False, needs_layout_passes=False),
    )(idx_t, tok, pos_t)


def kernel(inputs, token_table, pos_table):
    idx_t = inputs.astype(jnp.int32).T
    out = _run(idx_t, token_table, pos_table.T)
    return jnp.transpose(out, (2, 0, 1))
